# baseline (speedup 1.0000x reference)
import functools

import jax
import jax.numpy as jnp
from jax import lax
from jax.experimental import pallas as pl
from jax.experimental.pallas import tpu as pltpu
from jax.experimental.pallas import tpu_sc as plsc

_N = 10000
_D = 128
_E = 320000
_LANES = 16
_NSUB = 16
_EPT = _E // _NSUB
_CHUNK = 80
_NCHUNK = _EPT // _CHUNK
_NBUF = 4
_REC = 3 * _CHUNK
_ROWS_PT = 624
_ZROWS = 48


def _mm_body(x_ref, w_ref, b_ref, o_ref):
    o_ref[...] = (
        jnp.dot(x_ref[...], w_ref[...], preferred_element_type=jnp.float32)
        + b_ref[...]
    )


def _mm(x, W, b):
    blk = 1000
    n = x.shape[0]
    return pl.pallas_call(
        _mm_body,
        grid=(n // blk,),
        in_specs=[
            pl.BlockSpec((blk, _D), lambda i: (i, 0)),
            pl.BlockSpec((_D, _D), lambda i: (0, 0)),
            pl.BlockSpec((1, _D), lambda i: (0, 0)),
        ],
        out_specs=pl.BlockSpec((blk, _D), lambda i: (i, 0)),
        out_shape=jax.ShapeDtypeStruct((n, _D), jnp.float32),
    )(x, W, b.reshape(1, _D))


def _l2_body(x_ref, o_ref):
    x = x_ref[...]
    n = jnp.sqrt(jnp.sum(x * x, axis=1, keepdims=True))
    o_ref[...] = x / jnp.maximum(n, 1e-12)


def _l2(x):
    blk = 1000
    n = x.shape[0]
    return pl.pallas_call(
        _l2_body,
        grid=(n // blk,),
        in_specs=[pl.BlockSpec((blk, _D), lambda i: (i, 0))],
        out_specs=pl.BlockSpec((blk, _D), lambda i: (i, 0)),
        out_shape=jax.ShapeDtypeStruct((n, _D), jnp.float32),
    )(x)


def _spmm_body(hp, epk, out,
               ebuf, dst_c, gbuf, zbuf, accum,
               sem_g, sem_s, sem_i):
    c = lax.axis_index("c")
    s = lax.axis_index("s")

    def zrow(i, _):
        for j in range(_D // _LANES):
            zbuf[i, pl.ds(j * _LANES, _LANES)] = jnp.zeros(
                (_LANES,), jnp.float32)
        return 0

    lax.fori_loop(0, _ZROWS, zrow, 0)
    row0 = s * _ROWS_PT
    for k in range(_ROWS_PT // _ZROWS):
        pltpu.sync_copy(zbuf, accum.at[pl.ds(row0 + k * _ZROWS, _ZROWS), :])
    pl.when(s == _NSUB - 1)(lambda: pltpu.sync_copy(
        zbuf.at[pl.ds(0, 16), :],
        accum.at[pl.ds(_NSUB * _ROWS_PT, 16), :]))
    plsc.subcore_barrier()

    base = (c * _NSUB + s) * _NCHUNK
    cn16 = jnp.broadcast_to(c * _N, (_LANES,)).astype(jnp.int32)

    def start_idx(j, p):
        pltpu.async_copy(epk.at[pl.ds((base + j) * _REC, _REC)], ebuf[p],
                         sem_i[p])

    def wait_idx(j, p):
        pltpu.make_async_copy(
            epk.at[pl.ds((base + j) * _REC, _REC)], ebuf[p], sem_i[p]).wait()
        for k in range(_CHUNK // _LANES):
            sl = pl.ds(k * _LANES, _LANES)
            ebuf[p][sl] = ebuf[p][sl] + cn16
            dst_c[p][sl] = ebuf[p][pl.ds(_CHUNK + k * _LANES, _LANES)]

    def start_gather(p):
        pltpu.async_copy(hp.at[ebuf[p].at[pl.ds(0, _CHUNK)]], gbuf[p],
                         sem_g[p])

    def wait_gather(p):
        pltpu.make_async_copy(hp.at[ebuf[p].at[pl.ds(0, _CHUNK)]], gbuf[p],
                              sem_g[p]).wait()

    def start_scatter(p):
        pltpu.async_copy(gbuf[p], accum.at[dst_c[p]], sem_s[p], add=True)

    def wait_scatter(p):
        pltpu.make_async_copy(gbuf[p], accum.at[dst_c[p]], sem_s[p]).wait()

    def scale(p):
        def grp_body(g, _):
            wv16 = lax.bitcast_convert_type(
                ebuf[p][pl.ds(2 * _CHUNK + g * _LANES, _LANES)], jnp.float32)
            for l in range(_LANES):
                wv = jnp.broadcast_to(wv16[l], (_LANES,))
                e = g * _LANES + l
                for q in range(_D // _LANES):
                    gbuf[p][e, pl.ds(q * _LANES, _LANES)] = (
                        gbuf[p][e, pl.ds(q * _LANES, _LANES)] * wv)
            return 0

        lax.fori_loop(0, _CHUNK // _LANES, grp_body, 0)

    start_idx(0, 0)
    start_idx(1, 1)
    wait_idx(0, 0)
    start_gather(0)
    wait_idx(1, 1)
    start_gather(1)

    def pipe_body(t, _):
        for b in range(_NBUF):
            j = t * _NBUF + b
            p = b
            p2 = (b + 2) % _NBUF
            wait_gather(p)
            pl.when(j >= 2)(lambda pp=p2: wait_scatter(pp))
            start_idx(j + 2, p2)
            wait_idx(j + 2, p2)
            start_gather(p2)
            scale(p)
            start_scatter(p)
        return 0

    lax.fori_loop(0, (_NCHUNK - 2) // _NBUF, pipe_body, 0)
    for j in (_NCHUNK - 2, _NCHUNK - 1):
        p = j % _NBUF
        wait_gather(p)
        wait_scatter((p + 2) % _NBUF)
        scale(p)
        start_scatter(p)
    wait_scatter((_NCHUNK - 2) % _NBUF)
    wait_scatter((_NCHUNK - 1) % _NBUF)
    plsc.subcore_barrier()

    out_row0 = c * _N + row0
    for k in range(_ROWS_PT // _ZROWS):
        pltpu.sync_copy(accum.at[pl.ds(row0 + k * _ZROWS, _ZROWS), :], zbuf)
        pltpu.sync_copy(zbuf, out.at[pl.ds(out_row0 + k * _ZROWS, _ZROWS), :])

    def tail():
        r = _NSUB * _ROWS_PT
        pltpu.sync_copy(accum.at[pl.ds(r, 16), :], zbuf.at[pl.ds(0, 16), :])
        pltpu.sync_copy(zbuf.at[pl.ds(0, 16), :],
                        out.at[pl.ds(c * _N + r, 16), :])

    pl.when(s == _NSUB - 1)(tail)


_spmm = functools.partial(
    pl.kernel,
    out_type=jax.ShapeDtypeStruct((2 * _N, _D), jnp.float32),
    mesh=plsc.VectorSubcoreMesh(core_axis_name="c", subcore_axis_name="s"),
    scratch_types=[
        [pltpu.VMEM((_REC,), jnp.int32)] * _NBUF,
        [pltpu.VMEM((_CHUNK,), jnp.int32)] * _NBUF,
        [pltpu.VMEM((_CHUNK, _D), jnp.float32)] * _NBUF,
        pltpu.VMEM((_ZROWS, _D), jnp.float32),
        pltpu.VMEM_SHARED((_N, _D), jnp.float32),
        [pltpu.SemaphoreType.DMA] * _NBUF,
        [pltpu.SemaphoreType.DMA] * _NBUF,
        [pltpu.SemaphoreType.DMA] * _NBUF,
    ],
)(_spmm_body)


def _pack_edges(ei, ew):
    src = ei[0].reshape(_E // _CHUNK, _CHUNK)
    dst = ei[1].reshape(_E // _CHUNK, _CHUNK)
    wb = jax.lax.bitcast_convert_type(ew, jnp.int32).reshape(
        _E // _CHUNK, _CHUNK)
    return jnp.concatenate([src, dst, wb], axis=1).reshape(-1)


def kernel(embedding1, embedding2, W0, b0, W1, b1,
           edge_index1, edge_weight1, edge_index2, edge_weight2):
    epk = jnp.concatenate([_pack_edges(edge_index1, edge_weight1),
                           _pack_edges(edge_index2, edge_weight2)])
    x = jnp.concatenate([embedding1, embedding2], axis=0)
    h1 = _mm(x, W0, b0)
    y = _spmm(h1, epk)
    h2 = _mm(y, W1, b1)
    z = _spmm(h2, epk)
    o = _l2(z)
    return o[:_N], o[_N:]


# speedup vs baseline: 1.0447x; 1.0447x over previous
import functools

import jax
import jax.numpy as jnp
from jax import lax
from jax.experimental import pallas as pl
from jax.experimental.pallas import tpu as pltpu
from jax.experimental.pallas import tpu_sc as plsc

_N = 10000
_D = 128
_E = 320000
_LANES = 16
_NSUB = 16
_EPT = _E // _NSUB
_CHUNK = 80
_NCHUNK = _EPT // _CHUNK
_NBUF = 4
_REC = 3 * _CHUNK
_ROWS_PT = 624
_ZROWS = 48


def _mm_body(x_ref, w_ref, b_ref, o_ref):
    o_ref[...] = (
        jnp.dot(x_ref[...], w_ref[...], preferred_element_type=jnp.float32)
        + b_ref[...]
    )


def _mm(x, W, b):
    blk = 1000
    n = x.shape[0]
    return pl.pallas_call(
        _mm_body,
        grid=(n // blk,),
        in_specs=[
            pl.BlockSpec((blk, _D), lambda i: (i, 0)),
            pl.BlockSpec((_D, _D), lambda i: (0, 0)),
            pl.BlockSpec((1, _D), lambda i: (0, 0)),
        ],
        out_specs=pl.BlockSpec((blk, _D), lambda i: (i, 0)),
        out_shape=jax.ShapeDtypeStruct((n, _D), jnp.float32),
    )(x, W, b.reshape(1, _D))


def _l2_body(x_ref, o_ref):
    x = x_ref[...]
    n = jnp.sqrt(jnp.sum(x * x, axis=1, keepdims=True))
    o_ref[...] = x / jnp.maximum(n, 1e-12)


def _l2(x):
    blk = 1000
    n = x.shape[0]
    return pl.pallas_call(
        _l2_body,
        grid=(n // blk,),
        in_specs=[pl.BlockSpec((blk, _D), lambda i: (i, 0))],
        out_specs=pl.BlockSpec((blk, _D), lambda i: (i, 0)),
        out_shape=jax.ShapeDtypeStruct((n, _D), jnp.float32),
    )(x)


def _spmm_body(hp, epk, out,
               ebuf, dst_c, gbuf, zbuf, accum,
               sem_g, sem_s, sem_i):
    c = lax.axis_index("c")
    s = lax.axis_index("s")

    def zrow(i, _):
        for j in range(_D // _LANES):
            zbuf[i, pl.ds(j * _LANES, _LANES)] = jnp.zeros(
                (_LANES,), jnp.float32)
        return 0

    lax.fori_loop(0, _ZROWS, zrow, 0)
    row0 = s * _ROWS_PT
    for k in range(_ROWS_PT // _ZROWS):
        pltpu.sync_copy(zbuf, accum.at[pl.ds(row0 + k * _ZROWS, _ZROWS), :])
    pl.when(s == _NSUB - 1)(lambda: pltpu.sync_copy(
        zbuf.at[pl.ds(0, 16), :],
        accum.at[pl.ds(_NSUB * _ROWS_PT, 16), :]))
    plsc.subcore_barrier()

    base = (c * _NSUB + s) * _NCHUNK
    cn16 = jnp.broadcast_to(c * _N, (_LANES,)).astype(jnp.int32)

    def start_idx(j, p):
        pltpu.async_copy(epk.at[pl.ds((base + j) * _REC, _REC)], ebuf[p],
                         sem_i[p])

    def wait_idx(j, p):
        pltpu.make_async_copy(
            epk.at[pl.ds((base + j) * _REC, _REC)], ebuf[p], sem_i[p]).wait()
        for k in range(_CHUNK // _LANES):
            sl = pl.ds(k * _LANES, _LANES)
            ebuf[p][sl] = ebuf[p][sl] + cn16
            dst_c[p][sl] = ebuf[p][pl.ds(_CHUNK + k * _LANES, _LANES)]

    def start_gather(p):
        pltpu.async_copy(hp.at[ebuf[p].at[pl.ds(0, _CHUNK)]], gbuf[p],
                         sem_g[p])

    def wait_gather(p):
        pltpu.make_async_copy(hp.at[ebuf[p].at[pl.ds(0, _CHUNK)]], gbuf[p],
                              sem_g[p]).wait()

    def start_scatter(p):
        pltpu.async_copy(gbuf[p], accum.at[dst_c[p]], sem_s[p], add=True)

    def wait_scatter(p):
        pltpu.make_async_copy(gbuf[p], accum.at[dst_c[p]], sem_s[p]).wait()

    def scale(p):
        def grp_body(g, _):
            wv16 = lax.bitcast_convert_type(
                ebuf[p][pl.ds(2 * _CHUNK + g * _LANES, _LANES)], jnp.float32)
            for l in range(_LANES):
                wv = jnp.broadcast_to(wv16[l], (_LANES,))
                e = g * _LANES + l
                for q in range(_D // _LANES):
                    gbuf[p][e, pl.ds(q * _LANES, _LANES)] = (
                        gbuf[p][e, pl.ds(q * _LANES, _LANES)] * wv)
            return 0

        lax.fori_loop(0, _CHUNK // _LANES, grp_body, 0)

    start_idx(0, 0)
    start_idx(1, 1)
    wait_idx(0, 0)
    start_gather(0)
    wait_idx(1, 1)
    start_gather(1)

    def pipe_body(t, _):
        for b in range(_NBUF):
            j = t * _NBUF + b
            p = b
            p2 = (b + 2) % _NBUF
            wait_gather(p)
            pl.when(j >= 2)(lambda pp=p2: wait_scatter(pp))
            start_idx(j + 2, p2)
            scale(p)
            start_scatter(p)
            wait_idx(j + 2, p2)
            start_gather(p2)
        return 0

    lax.fori_loop(0, (_NCHUNK - 2) // _NBUF, pipe_body, 0)
    for j in (_NCHUNK - 2, _NCHUNK - 1):
        p = j % _NBUF
        wait_gather(p)
        wait_scatter((p + 2) % _NBUF)
        scale(p)
        start_scatter(p)
    wait_scatter((_NCHUNK - 2) % _NBUF)
    wait_scatter((_NCHUNK - 1) % _NBUF)
    plsc.subcore_barrier()

    out_row0 = c * _N + row0
    for k in range(_ROWS_PT // _ZROWS):
        pltpu.sync_copy(accum.at[pl.ds(row0 + k * _ZROWS, _ZROWS), :], zbuf)
        pltpu.sync_copy(zbuf, out.at[pl.ds(out_row0 + k * _ZROWS, _ZROWS), :])

    def tail():
        r = _NSUB * _ROWS_PT
        pltpu.sync_copy(accum.at[pl.ds(r, 16), :], zbuf.at[pl.ds(0, 16), :])
        pltpu.sync_copy(zbuf.at[pl.ds(0, 16), :],
                        out.at[pl.ds(c * _N + r, 16), :])

    pl.when(s == _NSUB - 1)(tail)


_spmm = functools.partial(
    pl.kernel,
    out_type=jax.ShapeDtypeStruct((2 * _N, _D), jnp.float32),
    mesh=plsc.VectorSubcoreMesh(core_axis_name="c", subcore_axis_name="s"),
    scratch_types=[
        [pltpu.VMEM((_REC,), jnp.int32)] * _NBUF,
        [pltpu.VMEM((_CHUNK,), jnp.int32)] * _NBUF,
        [pltpu.VMEM((_CHUNK, _D), jnp.float32)] * _NBUF,
        pltpu.VMEM((_ZROWS, _D), jnp.float32),
        pltpu.VMEM_SHARED((_N, _D), jnp.float32),
        [pltpu.SemaphoreType.DMA] * _NBUF,
        [pltpu.SemaphoreType.DMA] * _NBUF,
        [pltpu.SemaphoreType.DMA] * _NBUF,
    ],
)(_spmm_body)


def _pack_edges(ei, ew):
    src = ei[0].reshape(_E // _CHUNK, _CHUNK)
    dst = ei[1].reshape(_E // _CHUNK, _CHUNK)
    wb = jax.lax.bitcast_convert_type(ew, jnp.int32).reshape(
        _E // _CHUNK, _CHUNK)
    return jnp.concatenate([src, dst, wb], axis=1).reshape(-1)


def kernel(embedding1, embedding2, W0, b0, W1, b1,
           edge_index1, edge_weight1, edge_index2, edge_weight2):
    epk = jnp.concatenate([_pack_edges(edge_index1, edge_weight1),
                           _pack_edges(edge_index2, edge_weight2)])
    x = jnp.concatenate([embedding1, embedding2], axis=0)
    h1 = _mm(x, W0, b0)
    y = _spmm(h1, epk)
    h2 = _mm(y, W1, b1)
    z = _spmm(h2, epk)
    o = _l2(z)
    return o[:_N], o[_N:]
